# baseline (speedup 1.0000x reference)
import functools

import jax
import jax.numpy as jnp
from jax.experimental import pallas as pl
from jax.experimental.pallas import tpu as pltpu

_EPS = 1e-12


def _qnorm(q):
    n = jnp.sqrt(jnp.sum(q * q, axis=1, keepdims=True))
    return q / jnp.maximum(n, _EPS)


def _tile_sims(q_ref, m_ref, tile_n, n_modes):
    j = pl.program_id(0)
    q = _qnorm(q_ref[...])
    m = m_ref[...]
    row = j * tile_n + jax.lax.broadcasted_iota(jnp.int32, m.shape, 0)
    m = jnp.where(row < n_modes, m, 0.0)
    mn_inv = 1.0 / jnp.maximum(jnp.sqrt(jnp.sum(m * m, axis=1)), _EPS)
    s = jax.lax.dot_general(q, m, (((1,), (1,)), ((), ())),
                            preferred_element_type=jnp.float32)
    s = s * mn_inv[None, :]
    col = j * tile_n + jax.lax.broadcasted_iota(jnp.int32, s.shape, 1)
    e = jnp.where(col < n_modes, jnp.exp(s - 1.0), 0.0)
    return e, m


def _sum_body(q_ref, m_ref, sum_ref, acc_ref, *, tile_n, n_modes, n_tiles):
    j = pl.program_id(0)
    e, _ = _tile_sims(q_ref, m_ref, tile_n, n_modes)

    @pl.when(j == 0)
    def _init():
        acc_ref[...] = jnp.zeros_like(acc_ref)

    acc_ref[...] += jnp.sum(e, axis=1, keepdims=True)

    @pl.when(j == n_tiles - 1)
    def _fin():
        sum_ref[...] = acc_ref[...]


def _write_body(q_ref, m_ref, sum_ref, w_ref, v_ref, acc_ref, *,
                tile_n, n_modes, n_tiles):
    j = pl.program_id(0)
    e, m = _tile_sims(q_ref, m_ref, tile_n, n_modes)
    w = e * (1.0 / sum_ref[...])
    w_ref[...] = w

    @pl.when(j == 0)
    def _init():
        acc_ref[...] = jnp.zeros_like(acc_ref)

    acc_ref[...] += jax.lax.dot_general(w, m, (((1,), (0,)), ((), ())),
                                        preferred_element_type=jnp.float32)

    @pl.when(j == n_tiles - 1)
    def _fin():
        v_ref[...] = acc_ref[...]


@functools.partial(jax.jit, static_argnames=("tile_n",))
def _run(query, modes, tile_n=1024):
    b, d = query.shape
    n = modes.shape[0]
    n_tiles = pl.cdiv(n, tile_n)

    sum_spec = pl.BlockSpec((b, 1), lambda j: (0, 0))
    q_spec = pl.BlockSpec((b, d), lambda j: (0, 0))
    m_spec = pl.BlockSpec((tile_n, d), lambda j: (j, 0))

    sumexp = pl.pallas_call(
        functools.partial(_sum_body, tile_n=tile_n, n_modes=n, n_tiles=n_tiles),
        grid=(n_tiles,),
        in_specs=[q_spec, m_spec],
        out_specs=sum_spec,
        out_shape=jax.ShapeDtypeStruct((b, 1), jnp.float32),
        scratch_shapes=[pltpu.VMEM((b, 1), jnp.float32)],
        compiler_params=pltpu.CompilerParams(
            dimension_semantics=("arbitrary",)),
    )(query, modes)

    weights, value = pl.pallas_call(
        functools.partial(_write_body, tile_n=tile_n, n_modes=n,
                          n_tiles=n_tiles),
        grid=(n_tiles,),
        in_specs=[q_spec, m_spec, sum_spec],
        out_specs=[pl.BlockSpec((b, tile_n), lambda j: (0, j)),
                   pl.BlockSpec((b, d), lambda j: (0, 0))],
        out_shape=[jax.ShapeDtypeStruct((b, n), jnp.float32),
                   jax.ShapeDtypeStruct((b, d), jnp.float32)],
        scratch_shapes=[pltpu.VMEM((b, d), jnp.float32)],
        compiler_params=pltpu.CompilerParams(
            dimension_semantics=("arbitrary",)),
    )(query, modes, sumexp)

    return value, weights


def kernel(query, modes):
    return _run(query, modes)
